# Initial kernel scaffold; baseline (speedup 1.0000x reference)
import jax
import jax.numpy as jnp
from jax.experimental import pallas as pl


def kernel(x, embedding, temporal_embedding):
    raise NotImplementedError("write your pallas kernel here")


# speedup vs baseline: 9.4053x; 9.4053x over previous
import numpy as np
import jax
import jax.numpy as jnp
from jax import lax
from jax.experimental import pallas as pl
from jax.experimental.pallas import tpu as pltpu
from jax.experimental.pallas import tpu_sc as plsc

_VOCAB = 100000
_D = 64
_S = 20
_M = 50
_B = 1024

_NC, _NS = 2, 16
_NW = _NC * _NS
_ROWS = _B * _M
_RPW = _ROWS // _NW
_CH = 40
_NCH = _RPW // _CH
_K = _S + 1
_G = _CH * _K
_GSUB = 120
_NGS = _G // _GSUB


def _weights():
    j = np.arange(1, _S + 1, dtype=np.float32)[:, None]
    k = np.arange(1, _D + 1, dtype=np.float32)[None, :]
    enc = (1.0 - j / _S) - (k / _D) * (1.0 - 2.0 * j / _S)
    return jnp.asarray(
        np.concatenate([enc, np.ones((1, _D), np.float32)], axis=0)
    )


def _body(idx_hbm, table_hbm, w_hbm, out_hbm, idx_v, rows_v, out_v, w_v, sem):
    wid = lax.axis_index("s") * _NC + lax.axis_index("c")
    base = wid * _RPW
    pltpu.sync_copy(w_hbm, w_v)

    def chunk(c, carry):
        row0 = base + c * _CH
        pltpu.sync_copy(idx_hbm.at[pl.ds(row0 * _K, _G)], idx_v)
        for j in range(_NGS):
            pltpu.make_async_copy(
                table_hbm.at[idx_v.at[pl.ds(j * _GSUB, _GSUB)]],
                rows_v.at[pl.ds(j * _GSUB, _GSUB)],
                sem,
            ).start()
        pltpu.make_async_copy(table_hbm.at[pl.ds(0, _G)], rows_v, sem).wait()

        def row(r, carry2):
            b0 = r * _K
            for jj in range(_D // 16):
                dsl = pl.ds(jj * 16, 16)
                acc = rows_v[b0, dsl] * w_v[0, dsl]
                for k in range(1, _K):
                    acc = acc + rows_v[b0 + k, dsl] * w_v[k, dsl]
                out_v[r, dsl] = acc
            return carry2

        lax.fori_loop(0, _CH, row, 0)
        pltpu.sync_copy(out_v, out_hbm.at[pl.ds(row0, _CH)])
        return carry

    lax.fori_loop(0, _NCH, chunk, 0)


def kernel(x, embedding, temporal_embedding):
    table = jnp.concatenate([embedding, temporal_embedding], axis=0)
    xi = x.reshape(_ROWS, _S)
    tslot = (jnp.arange(_ROWS, dtype=jnp.int32) % _M + _VOCAB)[:, None]
    idx = jnp.concatenate([xi, tslot], axis=1).reshape(-1)
    mesh = plsc.VectorSubcoreMesh(core_axis_name="c", subcore_axis_name="s")
    out = pl.kernel(
        _body,
        mesh=mesh,
        compiler_params=pltpu.CompilerParams(use_tc_tiling_on_sc=False),
        out_type=jax.ShapeDtypeStruct((_ROWS, _D), jnp.float32),
        scratch_types=[
            pltpu.VMEM((_G,), jnp.int32),
            pltpu.VMEM((_G, _D), jnp.float32),
            pltpu.VMEM((_CH, _D), jnp.float32),
            pltpu.VMEM((_K, _D), jnp.float32),
            pltpu.SemaphoreType.DMA,
        ],
    )(idx, table, _weights())
    return out.reshape(_B, _M, _D)


# speedup vs baseline: 11.6452x; 1.2382x over previous
import numpy as np
import jax
import jax.numpy as jnp
from jax import lax
from jax.experimental import pallas as pl
from jax.experimental.pallas import tpu as pltpu
from jax.experimental.pallas import tpu_sc as plsc

_VOCAB = 100000
_D = 64
_S = 20
_M = 50
_B = 1024

_NC, _NS = 2, 16
_NW = _NC * _NS
_ROWS = _B * _M
_RPW = _ROWS // _NW
_CH = 40
_NCH = _RPW // _CH
_K = _S + 1
_G = _CH * _K
_GSUB = 120
_NGS = _G // _GSUB


def _weights():
    j = np.arange(1, _S + 1, dtype=np.float32)[:, None]
    k = np.arange(1, _D + 1, dtype=np.float32)[None, :]
    enc = (1.0 - j / _S) - (k / _D) * (1.0 - 2.0 * j / _S)
    return jnp.asarray(
        np.concatenate([enc, np.ones((1, _D), np.float32)], axis=0)
    )


def _body(idx_hbm, table_hbm, w_hbm, out_hbm,
          idx0, idx1, rows0, rows1, out_v, w_v, sem0, sem1):
    wid = lax.axis_index("s") * _NC + lax.axis_index("c")
    base = wid * _RPW
    pltpu.sync_copy(w_hbm, w_v)
    idx_b, rows_b, sem_b = (idx0, idx1), (rows0, rows1), (sem0, sem1)

    def fire(c, b):
        pltpu.sync_copy(idx_hbm.at[pl.ds((base + c * _CH) * _K, _G)], idx_b[b])
        for j in range(_NGS):
            pltpu.make_async_copy(
                table_hbm.at[idx_b[b].at[pl.ds(j * _GSUB, _GSUB)]],
                rows_b[b].at[pl.ds(j * _GSUB, _GSUB)],
                sem_b[b],
            ).start()

    def drain(b):
        pltpu.make_async_copy(
            table_hbm.at[pl.ds(0, _G)], rows_b[b], sem_b[b]
        ).wait()

    def compute(c, b):
        rows_v = rows_b[b]

        def row(r, carry2):
            b0 = r * _K
            for jj in range(_D // 16):
                dsl = pl.ds(jj * 16, 16)
                acc = rows_v[b0, dsl] * w_v[0, dsl]
                for k in range(1, _K):
                    acc = acc + rows_v[b0 + k, dsl] * w_v[k, dsl]
                out_v[r, dsl] = acc
            return carry2

        lax.fori_loop(0, _CH, row, 0)
        pltpu.sync_copy(out_v, out_hbm.at[pl.ds(base + c * _CH, _CH)])

    fire(0, 0)
    fire(1, 1)

    def pair(i, carry):
        for b in range(2):
            c = i * 2 + b
            drain(b)
            compute(c, b)

            @pl.when(c + 2 < _NCH)
            def _():
                fire(c + 2, b)
        return carry

    lax.fori_loop(0, _NCH // 2, pair, 0)


def kernel(x, embedding, temporal_embedding):
    table = jnp.concatenate([embedding, temporal_embedding], axis=0)
    xi = x.reshape(_ROWS, _S)
    tslot = (jnp.arange(_ROWS, dtype=jnp.int32) % _M + _VOCAB)[:, None]
    idx = jnp.concatenate([xi, tslot], axis=1).reshape(-1)
    mesh = plsc.VectorSubcoreMesh(core_axis_name="c", subcore_axis_name="s")
    out = pl.kernel(
        _body,
        mesh=mesh,
        compiler_params=pltpu.CompilerParams(use_tc_tiling_on_sc=False),
        out_type=jax.ShapeDtypeStruct((_ROWS, _D), jnp.float32),
        scratch_types=[
            pltpu.VMEM((_G,), jnp.int32),
            pltpu.VMEM((_G,), jnp.int32),
            pltpu.VMEM((_G, _D), jnp.float32),
            pltpu.VMEM((_G, _D), jnp.float32),
            pltpu.VMEM((_CH, _D), jnp.float32),
            pltpu.VMEM((_K, _D), jnp.float32),
            pltpu.SemaphoreType.DMA,
            pltpu.SemaphoreType.DMA,
        ],
    )(idx, table, _weights())
    return out.reshape(_B, _M, _D)


# speedup vs baseline: 13.3150x; 1.1434x over previous
import numpy as np
import jax
import jax.numpy as jnp
from jax import lax
from jax.experimental import pallas as pl
from jax.experimental.pallas import tpu as pltpu
from jax.experimental.pallas import tpu_sc as plsc

_D = 64
_S = 20
_M = 50
_B = 1024

_NC, _NS = 2, 16
_NW = _NC * _NS
_ROWS = _B * _M
_RPW = _ROWS // _NW
_CH = 40
_NCH = _RPW // _CH
_G = _CH * _S
_GPART = [(0, 128), (128, 128), (256, 128), (384, 128),
          (512, 128), (640, 128), (768, 32)]


def _pos_enc():
    j = np.arange(1, _S + 1, dtype=np.float32)[:, None]
    k = np.arange(1, _D + 1, dtype=np.float32)[None, :]
    return jnp.asarray((1.0 - j / _S) - (k / _D) * (1.0 - 2.0 * j / _S))


def _body(x_hbm, table_hbm, te_hbm, w_hbm, out_hbm,
          idx0, idx1, rows0, rows1, out_v, te_v, w_v, sem0, sem1):
    wid = lax.axis_index("s") * _NC + lax.axis_index("c")
    base = wid * _RPW
    pltpu.sync_copy(w_hbm, w_v)
    pltpu.sync_copy(te_hbm, te_v)
    idx_b, rows_b, sem_b = (idx0, idx1), (rows0, rows1), (sem0, sem1)

    def fire(c, b):
        pltpu.sync_copy(x_hbm.at[pl.ds((base + c * _CH) * _S, _G)], idx_b[b])
        for off, sz in _GPART:
            pltpu.make_async_copy(
                table_hbm.at[idx_b[b].at[pl.ds(off, sz)]],
                rows_b[b].at[pl.ds(off, sz)],
                sem_b[b],
            ).start()

    def drain(b):
        pltpu.make_async_copy(
            table_hbm.at[pl.ds(0, _G)], rows_b[b], sem_b[b]
        ).wait()

    def compute(c, b):
        rows_v = rows_b[b]
        m0 = lax.rem(c * _CH, _M)

        def row(r, carry2):
            b0 = r * _S
            mr = lax.rem(m0 + r, _M)
            for jj in range(_D // 16):
                dsl = pl.ds(jj * 16, 16)
                acc = te_v[mr, dsl]
                for k in range(_S):
                    acc = acc + rows_v[b0 + k, dsl] * w_v[k, dsl]
                out_v[r, dsl] = acc
            return carry2

        lax.fori_loop(0, _CH, row, 0)
        pltpu.sync_copy(out_v, out_hbm.at[pl.ds(base + c * _CH, _CH)])

    fire(0, 0)
    fire(1, 1)

    def pair(i, carry):
        for b in range(2):
            c = i * 2 + b
            drain(b)
            compute(c, b)

            @pl.when(c + 2 < _NCH)
            def _():
                fire(c + 2, b)
        return carry

    lax.fori_loop(0, _NCH // 2, pair, 0)


def kernel(x, embedding, temporal_embedding):
    mesh = plsc.VectorSubcoreMesh(core_axis_name="c", subcore_axis_name="s")
    out = pl.kernel(
        _body,
        mesh=mesh,
        compiler_params=pltpu.CompilerParams(use_tc_tiling_on_sc=False),
        out_type=jax.ShapeDtypeStruct((_ROWS, _D), jnp.float32),
        scratch_types=[
            pltpu.VMEM((_G,), jnp.int32),
            pltpu.VMEM((_G,), jnp.int32),
            pltpu.VMEM((_G, _D), jnp.float32),
            pltpu.VMEM((_G, _D), jnp.float32),
            pltpu.VMEM((_CH, _D), jnp.float32),
            pltpu.VMEM((_M, _D), jnp.float32),
            pltpu.VMEM((_S, _D), jnp.float32),
            pltpu.SemaphoreType.DMA,
            pltpu.SemaphoreType.DMA,
        ],
    )(x.reshape(-1), embedding, temporal_embedding, _pos_enc())
    return out.reshape(_B, _M, _D)
